# Initial kernel scaffold; baseline (speedup 1.0000x reference)
import jax
import jax.numpy as jnp
from jax.experimental import pallas as pl


def kernel(x, c, notj, edge_index, e1, edge_index_P, index_m, PT, W_v, W_v2, W_v3, v, m):
    raise NotImplementedError("write your pallas kernel here")


# speedup vs baseline: 159.8932x; 159.8932x over previous
import jax
import jax.numpy as jnp
from jax import lax
from jax.experimental import pallas as pl
from jax.experimental.pallas import tpu as pltpu
from jax.experimental.pallas import tpu_sc as plsc

f32 = jnp.float32
i32 = jnp.int32

CH = 2000
RB = 128
RR = 16
CHM = RB * RR
ZC = 1600


def _sc_call(N, Ep, EPp, NCHM, NC, NS, mesh):
    NW = NC * NS
    nchE_w = Ep // CH // NW
    nchP_w = EPp // CH // NW
    nchB = NCHM // NS
    nchC = NCHM // NW
    stripe0 = -(-N // NS)
    stripe = -(-stripe0 // ZC) * ZC
    Npad = NS * stripe
    nz = stripe // ZC

    def body(c_hbm, srcE_hbm, dstE_hbm, srcP_hbm, dstP_hbm,
             a3_hbm, b3_hbm, pt3_hbm, v_hbm,
             out1, out2, outs,
             c_tab, bi0, bi1, a2d, b2d, pt2d, msg2d, zf, vbuf, accbuf,
             em2_sh):
        cid = lax.axis_index("c")
        sid = lax.axis_index("s")
        wid = sid * NC + cid

        pltpu.sync_copy(v_hbm, vbuf)
        v_vec = vbuf[...]
        pltpu.sync_copy(c_hbm, c_tab)

        def edge_scan(s_hbm, d_hbm, nch_w):
            def outer(t, acc):
                base = (wid * nch_w + t) * CH
                pltpu.sync_copy(s_hbm.at[pl.ds(base, CH)], bi0)
                pltpu.sync_copy(d_hbm.at[pl.ds(base, CH)], bi1)

                def inner(j, acc):
                    sv = bi0[pl.ds(j * 16, 16)]
                    dv = bi1[pl.ds(j * 16, 16)]
                    cv = plsc.load_gather(c_tab, [sv])
                    return acc + jnp.where(dv == v_vec, cv, 0.0)

                return lax.fori_loop(0, CH // 16, inner, acc)

            return lax.fori_loop(0, nch_w, outer, jnp.zeros((16,), f32))

        acc1 = edge_scan(srcE_hbm, dstE_hbm, nchE_w)
        accbuf[...] = acc1
        pltpu.sync_copy(accbuf, out1.at[wid])

        acc2 = edge_scan(srcP_hbm, dstP_hbm, nchP_w)
        accbuf[...] = acc2
        pltpu.sync_copy(accbuf, out2.at[wid])

        def zb(i, _):
            zf[pl.ds(i * 16, 16)] = jnp.zeros((16,), f32)
            return 0

        lax.fori_loop(0, ZC // 16, zb, 0)
        for i in range(nz):
            pltpu.sync_copy(zf, em2_sh.at[pl.ds(sid * stripe + i * ZC, ZC)])
        plsc.subcore_barrier()

        def bouter(t, _):
            ch = sid * nchB + t
            pltpu.sync_copy(a3_hbm.at[ch], a2d)
            pltpu.sync_copy(b3_hbm.at[ch], b2d)
            pltpu.sync_copy(pt3_hbm.at[ch], pt2d)
            for r in range(RR):
                def mb(q, _, r=r):
                    bv = b2d[r, pl.ds(q * 16, 16)]
                    pv = pt2d[r, pl.ds(q * 16, 16)]
                    msg2d[r, pl.ds(q * 16, 16)] = jnp.where(bv == v_vec, 0.0, pv)
                    return 0

                lax.fori_loop(0, RB // 16, mb, 0)
            for r in range(RR):
                pltpu.sync_copy(msg2d.at[r], em2_sh.at[a2d.at[r]], add=True)
            return 0

        lax.fori_loop(0, nchB, bouter, 0)
        plsc.subcore_barrier()

        def couter(t, acc):
            ch = wid * nchC + t
            pltpu.sync_copy(a3_hbm.at[ch], a2d)
            pltpu.sync_copy(b3_hbm.at[ch], b2d)
            for r in range(RR):
                pltpu.sync_copy(em2_sh.at[a2d.at[r]], pt2d.at[r])
            for r in range(RR):
                def ci(q, a, r=r):
                    bv = b2d[r, pl.ds(q * 16, 16)]
                    gv = pt2d[r, pl.ds(q * 16, 16)]
                    return a + jnp.where(bv == v_vec, gv, 0.0)

                acc = lax.fori_loop(0, RB // 16, ci, acc)
            return acc

        accS = lax.fori_loop(0, nchC, couter, jnp.zeros((16,), f32))
        accbuf[...] = accS
        pltpu.sync_copy(accbuf, outs.at[wid])

    sds = jax.ShapeDtypeStruct
    return pl.kernel(
        body,
        out_type=(sds((NW, 16), f32), sds((NW, 16), f32), sds((NW, 16), f32)),
        mesh=mesh,
        compiler_params=pltpu.CompilerParams(needs_layout_passes=False),
        scratch_types=[
            pltpu.VMEM((N,), f32),
            pltpu.VMEM((CH,), i32),
            pltpu.VMEM((CH,), i32),
            pltpu.VMEM((RR, RB), i32),
            pltpu.VMEM((RR, RB), i32),
            pltpu.VMEM((RR, RB), f32),
            pltpu.VMEM((RR, RB), f32),
            pltpu.VMEM((ZC,), f32),
            pltpu.VMEM((16,), i32),
            pltpu.VMEM((16,), f32),
            pltpu.VMEM_SHARED((Npad,), f32),
        ],
    )


def _tc_body(p1, p2, ps, wv, w2, w3, e1r, out):
    c1v = jnp.sum(p1[...])
    c2v = jnp.sum(p2[...])
    s = jnp.sum(ps[...])
    row = s * wv[...]
    h = jnp.maximum(jnp.dot(row, w2[...], preferred_element_type=f32), 0.0)
    v3 = jnp.dot(h, w3[...], preferred_element_type=f32)
    out[...] = v3 + e1r[...] + jnp.maximum(c1v, c2v)


def kernel(x, c, notj, edge_index, e1, edge_index_P, index_m, PT,
           W_v, W_v2, W_v3, v, m):
    N = c.shape[0]
    E = edge_index.shape[1]
    EP = edge_index_P.shape[1]
    EM = index_m.shape[0]

    mesh = plsc.VectorSubcoreMesh(core_axis_name="c", subcore_axis_name="s")
    NC, NS = mesh.num_cores, mesh.num_subcores
    NW = NC * NS

    def pad_pair(src, dst, tot):
        per = CH * NW
        totp = -(-tot // per) * per
        if totp != tot:
            padn = totp - tot
            src = jnp.concatenate([src, jnp.zeros((padn,), i32)])
            dst = jnp.concatenate([dst, jnp.full((padn,), -1, i32)])
        return src, dst, totp

    srcE, dstE, Ep = pad_pair(edge_index[0], edge_index[1], E)
    srcP, dstP, EPp = pad_pair(edge_index_P[0], edge_index_P[1], EP)

    NCHM = -(-(-(-EM // CHM)) // NW) * NW
    EMp = NCHM * CHM
    padn = EMp - EM
    a = index_m[:, 0]
    b = index_m[:, 1]
    pt = PT[:, 0]
    if padn:
        a = jnp.concatenate([a, jnp.zeros((padn,), i32)])
        b = jnp.concatenate([b, jnp.full((padn,), -1, i32)])
        pt = jnp.concatenate([pt, jnp.zeros((padn,), f32)])
    a3 = a.reshape(NCHM, RR, RB)
    b3 = b.reshape(NCHM, RR, RB)
    pt3 = pt.reshape(NCHM, RR, RB)

    v16 = jnp.full((16,), v, i32)
    cflat = c[:, 0]

    p1, p2, ps = _sc_call(N, Ep, EPp, NCHM, NC, NS, mesh)(
        cflat, srcE, dstE, srcP, dstP, a3, b3, pt3, v16)

    q = pl.pallas_call(
        _tc_body,
        out_shape=jax.ShapeDtypeStruct((1, 1), f32),
    )(p1, p2, ps, W_v, W_v2, W_v3, e1.reshape(1, 1))
    return q.reshape(1)


# speedup vs baseline: 247.7661x; 1.5496x over previous
import jax
import jax.numpy as jnp
from jax import lax
from jax.experimental import pallas as pl
from jax.experimental.pallas import tpu as pltpu
from jax.experimental.pallas import tpu_sc as plsc

f32 = jnp.float32
i32 = jnp.int32

ROW = 128
CHA_ROWS = 100
CHA = CHA_ROWS * ROW
CHM_ROWS = 32
CHM = CHM_ROWS * ROW
ZC = 1600


def _sc_call(N, Ep, EPp, MR, NC, NS, mesh):
    NW = NC * NS
    nchE = Ep // CHA
    nchP = EPp // CHA
    pairsE = (-(-nchE // NW) + 1) // 2
    pairsP = (-(-nchP // NW) + 1) // 2
    nchB_t = MR // CHM_ROWS // NS
    nchC_w = MR // CHM_ROWS // NW
    stripe0 = -(-N // NS)
    stripe = -(-stripe0 // ZC) * ZC
    Npad = NS * stripe
    nz = stripe // ZC

    def body(c_hbm, dstE_h, srcE_h, dstP_h, srcP_h, a2_h, b_h, pt_h, v_h,
             out1, out2, outs,
             dbuf0, dbuf1, srcrow, crow, abuf, bbuf, ptbuf, msgbuf, zf,
             vbuf, accv, semA, semB, semS, em2_sh):
        cid = lax.axis_index("c")
        sid = lax.axis_index("s")
        wid = sid * NC + cid

        pltpu.sync_copy(v_h, vbuf)
        v_vec = vbuf[...]
        zero16 = jnp.zeros((16,), f32)
        false16 = jnp.zeros((16,), jnp.bool_)

        def zb(i, _):
            zf[pl.ds(i * 16, 16)] = zero16
            return 0

        lax.fori_loop(0, ZC // 16, zb, 0)
        for i in range(nz):
            pltpu.sync_copy(zf, em2_sh.at[pl.ds(sid * stripe + i * ZC, ZC)])

        accv[...] = zero16

        def scan_chunk(buf, n16):
            def it(i, mor):
                dv = buf[pl.ds(i * 16, 16)]
                return jnp.logical_or(mor, dv == v_vec)

            return lax.fori_loop(0, n16, it, false16)

        def masked_rowsum(buf, off, valrow):
            def acc8(q, s):
                dv = buf[pl.ds(off + q * 16, 16)]
                cv = valrow[0, pl.ds(q * 16, 16)]
                return s + jnp.where(dv == v_vec, cv, 0.0)

            return lax.fori_loop(0, ROW // 16, acc8, zero16)

        def slow_rows(s_hbm, buf, ch):
            def rb(r, _):
                off = r * ROW

                def sc8(q, m):
                    dv = buf[pl.ds(off + q * 16, 16)]
                    return jnp.logical_or(m, dv == v_vec)

                mor = lax.fori_loop(0, ROW // 16, sc8, false16)

                @pl.when(jnp.any(mor))
                def _():
                    g = ch * CHA + off
                    pltpu.sync_copy(s_hbm.at[pl.ds(g, ROW)], srcrow.at[0])
                    pltpu.sync_copy(c_hbm.at[srcrow.at[0]], crow.at[0])
                    accv[...] = accv[...] + masked_rowsum(buf, off, crow)

                return 0

            lax.fori_loop(0, CHA_ROWS, rb, 0)

        def edge_phase(d_hbm, s_hbm, nch, pairs):
            def sl(ch):
                return d_hbm.at[pl.ds(ch * CHA, CHA)]

            pltpu.async_copy(sl(wid), dbuf0, semA)
            bufs = ((dbuf0, semA), (dbuf1, semB))

            def pair(tp, _):
                for par in range(2):
                    buf, sem = bufs[par]
                    obuf, osem = bufs[1 - par]
                    t = 2 * tp + par
                    ch = wid + NW * t

                    @pl.when(ch < nch)
                    def _(buf=buf, sem=sem, obuf=obuf, osem=osem, ch=ch):
                        pltpu.make_async_copy(sl(ch), buf, sem).wait()
                        chn = ch + NW

                        @pl.when(chn < nch)
                        def _():
                            pltpu.async_copy(sl(chn), obuf, osem)

                        mor = scan_chunk(buf, CHA // 16)

                        @pl.when(jnp.any(mor))
                        def _():
                            slow_rows(s_hbm, buf, ch)

                return 0

            lax.fori_loop(0, pairs, pair, 0)

        edge_phase(dstE_h, srcE_h, nchE, pairsE)
        pltpu.sync_copy(accv, out1.at[wid])
        accv[...] = zero16

        edge_phase(dstP_h, srcP_h, nchP, pairsP)
        pltpu.sync_copy(accv, out2.at[wid])
        accv[...] = zero16

        plsc.subcore_barrier()

        def bchunk(k, _):
            r0 = (sid * nchB_t + k) * CHM_ROWS
            pltpu.sync_copy(a2_h.at[pl.ds(r0, CHM_ROWS)], abuf)
            pltpu.sync_copy(b_h.at[pl.ds(r0 * ROW, CHM)], bbuf)
            pltpu.sync_copy(pt_h.at[pl.ds(r0 * ROW, CHM)], ptbuf)
            anym = jnp.any(scan_chunk(bbuf, CHM // 16))

            @pl.when(anym)
            def _():
                def mit(i, _):
                    bv = bbuf[pl.ds(i * 16, 16)]
                    pv = ptbuf[pl.ds(i * 16, 16)]
                    msgbuf[pl.ds(i * 16, 16)] = jnp.where(bv == v_vec, 0.0, pv)
                    return 0

                lax.fori_loop(0, CHM // 16, mit, 0)
                descs = [
                    pltpu.async_copy(msgbuf.at[pl.ds(r * ROW, ROW)],
                                     em2_sh.at[abuf.at[r]], semS, add=True)
                    for r in range(CHM_ROWS)
                ]
                for d in descs:
                    d.wait()

            @pl.when(jnp.logical_not(anym))
            def _():
                descs = [
                    pltpu.async_copy(ptbuf.at[pl.ds(r * ROW, ROW)],
                                     em2_sh.at[abuf.at[r]], semS, add=True)
                    for r in range(CHM_ROWS)
                ]
                for d in descs:
                    d.wait()

            return 0

        lax.fori_loop(0, nchB_t, bchunk, 0)
        plsc.subcore_barrier()

        def cchunk(k, _):
            r0 = (wid * nchC_w + k) * CHM_ROWS
            pltpu.sync_copy(b_h.at[pl.ds(r0 * ROW, CHM)], bbuf)
            mor = scan_chunk(bbuf, CHM // 16)

            @pl.when(jnp.any(mor))
            def _():
                def rb(r, _):
                    off = r * ROW

                    def sc8(q, m):
                        bv = bbuf[pl.ds(off + q * 16, 16)]
                        return jnp.logical_or(m, bv == v_vec)

                    mr = lax.fori_loop(0, ROW // 16, sc8, false16)

                    @pl.when(jnp.any(mr))
                    def _():
                        pltpu.sync_copy(a2_h.at[r0 + r], srcrow.at[0])
                        pltpu.sync_copy(em2_sh.at[srcrow.at[0]], crow.at[0])
                        accv[...] = accv[...] + masked_rowsum(bbuf, off, crow)

                    return 0

                lax.fori_loop(0, CHM_ROWS, rb, 0)

            return 0

        lax.fori_loop(0, nchC_w, cchunk, 0)
        pltpu.sync_copy(accv, outs.at[wid])

    sds = jax.ShapeDtypeStruct
    return pl.kernel(
        body,
        out_type=(sds((NW, 16), f32), sds((NW, 16), f32), sds((NW, 16), f32)),
        mesh=mesh,
        compiler_params=pltpu.CompilerParams(needs_layout_passes=False),
        scratch_types=[
            pltpu.VMEM((CHA,), i32),
            pltpu.VMEM((CHA,), i32),
            pltpu.VMEM((1, ROW), i32),
            pltpu.VMEM((1, ROW), f32),
            pltpu.VMEM((CHM_ROWS, ROW), i32),
            pltpu.VMEM((CHM,), i32),
            pltpu.VMEM((CHM,), f32),
            pltpu.VMEM((CHM,), f32),
            pltpu.VMEM((ZC,), f32),
            pltpu.VMEM((16,), i32),
            pltpu.VMEM((16,), f32),
            pltpu.SemaphoreType.DMA,
            pltpu.SemaphoreType.DMA,
            pltpu.SemaphoreType.DMA,
            pltpu.VMEM_SHARED((Npad,), f32),
        ],
    )


def _tc_body(p1, p2, ps, wv, w2, w3, e1r, out):
    c1v = jnp.sum(p1[...])
    c2v = jnp.sum(p2[...])
    s = jnp.sum(ps[...])
    row = s * wv[...]
    h = jnp.maximum(jnp.dot(row, w2[...], preferred_element_type=f32), 0.0)
    v3 = jnp.dot(h, w3[...], preferred_element_type=f32)
    out[...] = v3 + e1r[...] + jnp.maximum(c1v, c2v)


def kernel(x, c, notj, edge_index, e1, edge_index_P, index_m, PT,
           W_v, W_v2, W_v3, v, m):
    N = c.shape[0]
    E = edge_index.shape[1]
    EP = edge_index_P.shape[1]
    EM = index_m.shape[0]

    mesh = plsc.VectorSubcoreMesh(core_axis_name="c", subcore_axis_name="s")
    NC, NS = mesh.num_cores, mesh.num_subcores
    NW = NC * NS

    def pad_pair(src, dst, tot):
        totp = -(-tot // CHA) * CHA
        if totp != tot:
            padn = totp - tot
            src = jnp.concatenate([src, jnp.zeros((padn,), i32)])
            dst = jnp.concatenate([dst, jnp.full((padn,), -1, i32)])
        return src, dst, totp

    srcE, dstE, Ep = pad_pair(edge_index[0], edge_index[1], E)
    srcP, dstP, EPp = pad_pair(edge_index_P[0], edge_index_P[1], EP)

    per_m = CHM * NW
    EMp = -(-EM // per_m) * per_m
    MR = EMp // ROW
    padn = EMp - EM
    a = index_m[:, 0]
    b = index_m[:, 1]
    pt = PT[:, 0]
    if padn:
        a = jnp.concatenate([a, jnp.zeros((padn,), i32)])
        b = jnp.concatenate([b, jnp.full((padn,), -1, i32)])
        pt = jnp.concatenate([pt, jnp.zeros((padn,), f32)])
    a2 = a.reshape(MR, ROW)

    v16 = jnp.full((16,), v, i32)
    cflat = c[:, 0]

    p1, p2, ps = _sc_call(N, Ep, EPp, MR, NC, NS, mesh)(
        cflat, dstE, srcE, dstP, srcP, a2, b, pt, v16)

    q = pl.pallas_call(
        _tc_body,
        out_shape=jax.ShapeDtypeStruct((1, 1), f32),
    )(p1, p2, ps, W_v, W_v2, W_v3, e1.reshape(1, 1))
    return q.reshape(1)


# speedup vs baseline: 325.3501x; 1.3131x over previous
import jax
import jax.numpy as jnp
from jax import lax
from jax.experimental import pallas as pl
from jax.experimental.pallas import tpu as pltpu
from jax.experimental.pallas import tpu_sc as plsc

f32 = jnp.float32
i32 = jnp.int32

ROW = 128
CHA_ROWS = 100
CHA = CHA_ROWS * ROW
CHM_ROWS = 32
CHM = CHM_ROWS * ROW
ZC = 1600


def _sc_call(N, Ep, EPp, MR, NC, NS, mesh):
    NW = NC * NS
    nchE = Ep // CHA
    nchP = EPp // CHA
    pairsE = (-(-nchE // NW) + 1) // 2
    pairsP = (-(-nchP // NW) + 1) // 2
    nchB_t = MR // CHM_ROWS // NS
    nchC_w = MR // CHM_ROWS // NW
    stripe0 = -(-N // NS)
    stripe = -(-stripe0 // ZC) * ZC
    Npad = NS * stripe
    nz = stripe // ZC

    def body(c_hbm, dstE_h, srcE_h, dstP_h, srcP_h, a2_h, b_h, pt_h, v_h,
             out1, out2, outs,
             dbuf0, dbuf1, srcrow, crow,
             abuf0, bbuf0, ptbuf0, msgbuf0,
             abuf1, bbuf1, ptbuf1, msgbuf1,
             zf, vbuf, accv,
             semA, semB, semS, semM0, semM1, em2_sh):
        cid = lax.axis_index("c")
        sid = lax.axis_index("s")
        wid = sid * NC + cid

        pltpu.sync_copy(v_h, vbuf)
        v_vec = vbuf[...]
        zero16 = jnp.zeros((16,), f32)
        false16 = jnp.zeros((16,), jnp.bool_)

        def zb(i, _):
            zf[pl.ds(i * 16, 16)] = zero16
            return 0

        lax.fori_loop(0, ZC // 16, zb, 0)
        for i in range(nz):
            pltpu.sync_copy(zf, em2_sh.at[pl.ds(sid * stripe + i * ZC, ZC)])
        plsc.subcore_barrier()

        def scan_chunk(buf, nrows):
            def it(r, mor):
                off = r * ROW
                for q in range(ROW // 16):
                    dv = buf[pl.ds(off + q * 16, 16)]
                    mor = jnp.logical_or(mor, dv == v_vec)
                return mor

            return lax.fori_loop(0, nrows, it, false16)

        def masked_rowsum(buf, off, valrow):
            def acc8(q, s):
                dv = buf[pl.ds(off + q * 16, 16)]
                cv = valrow[0, pl.ds(q * 16, 16)]
                return s + jnp.where(dv == v_vec, cv, 0.0)

            return lax.fori_loop(0, ROW // 16, acc8, zero16)

        bsets = ((abuf0, bbuf0, ptbuf0, msgbuf0, semM0),
                 (abuf1, bbuf1, ptbuf1, msgbuf1, semM1))

        def b_copies(k, st):
            ab, bb, ptb, _, sm = st
            r0 = (sid * nchB_t + k) * CHM_ROWS
            return ((a2_h.at[pl.ds(r0, CHM_ROWS)], ab, sm),
                    (b_h.at[pl.ds(r0 * ROW, CHM)], bb, sm),
                    (pt_h.at[pl.ds(r0 * ROW, CHM)], ptb, sm))

        def drain_scatters(st):
            ab, _, ptb, _, _ = st
            for r in range(CHM_ROWS):
                pltpu.make_async_copy(ptb.at[pl.ds(r * ROW, ROW)],
                                      em2_sh.at[ab.at[r]], semS).wait()

        for c3 in b_copies(0, bsets[0]):
            pltpu.async_copy(*c3)

        def bpair(tp, _):
            for par in range(2):
                cur = bsets[par]
                oth = bsets[1 - par]
                k = 2 * tp + par
                for c3 in b_copies(k, cur):
                    pltpu.make_async_copy(*c3).wait()

                @pl.when(k >= 1)
                def _(oth=oth):
                    drain_scatters(oth)

                @pl.when(k + 1 < nchB_t)
                def _(k=k, oth=oth):
                    for c3 in b_copies(k + 1, oth):
                        pltpu.async_copy(*c3)

                ab, bb, ptb, mb, _ = cur
                anym = jnp.any(scan_chunk(bb, CHM_ROWS))

                @pl.when(anym)
                def _(ab=ab, bb=bb, ptb=ptb, mb=mb):
                    def mit(i, _):
                        bv = bb[pl.ds(i * 16, 16)]
                        pv = ptb[pl.ds(i * 16, 16)]
                        mb[pl.ds(i * 16, 16)] = jnp.where(bv == v_vec, 0.0, pv)
                        return 0

                    lax.fori_loop(0, CHM // 16, mit, 0)
                    for r in range(CHM_ROWS):
                        pltpu.async_copy(mb.at[pl.ds(r * ROW, ROW)],
                                         em2_sh.at[ab.at[r]], semS, add=True)

                @pl.when(jnp.logical_not(anym))
                def _(ab=ab, ptb=ptb):
                    for r in range(CHM_ROWS):
                        pltpu.async_copy(ptb.at[pl.ds(r * ROW, ROW)],
                                         em2_sh.at[ab.at[r]], semS, add=True)

            return 0

        lax.fori_loop(0, nchB_t // 2, bpair, 0)
        drain_scatters(bsets[1])

        accv[...] = zero16

        def slow_rows(s_hbm, buf, ch):
            def rb(r, _):
                off = r * ROW

                def sc8(q, m):
                    dv = buf[pl.ds(off + q * 16, 16)]
                    return jnp.logical_or(m, dv == v_vec)

                mor = lax.fori_loop(0, ROW // 16, sc8, false16)

                @pl.when(jnp.any(mor))
                def _():
                    g = ch * CHA + off
                    pltpu.sync_copy(s_hbm.at[pl.ds(g, ROW)], srcrow.at[0])
                    pltpu.sync_copy(c_hbm.at[srcrow.at[0]], crow.at[0])
                    accv[...] = accv[...] + masked_rowsum(buf, off, crow)

                return 0

            lax.fori_loop(0, CHA_ROWS, rb, 0)

        def edge_phase(d_hbm, s_hbm, nch, pairs):
            def sl(ch):
                return d_hbm.at[pl.ds(ch * CHA, CHA)]

            pltpu.async_copy(sl(wid), dbuf0, semA)
            bufs = ((dbuf0, semA), (dbuf1, semB))

            def pair(tp, _):
                for par in range(2):
                    buf, sem = bufs[par]
                    obuf, osem = bufs[1 - par]
                    t = 2 * tp + par
                    ch = wid + NW * t

                    @pl.when(ch < nch)
                    def _(buf=buf, sem=sem, obuf=obuf, osem=osem, ch=ch):
                        pltpu.make_async_copy(sl(ch), buf, sem).wait()
                        chn = ch + NW

                        @pl.when(chn < nch)
                        def _():
                            pltpu.async_copy(sl(chn), obuf, osem)

                        mor = scan_chunk(buf, CHA_ROWS)

                        @pl.when(jnp.any(mor))
                        def _():
                            slow_rows(s_hbm, buf, ch)

                return 0

            lax.fori_loop(0, pairs, pair, 0)

        edge_phase(dstE_h, srcE_h, nchE, pairsE)
        pltpu.sync_copy(accv, out1.at[wid])
        accv[...] = zero16

        edge_phase(dstP_h, srcP_h, nchP, pairsP)
        pltpu.sync_copy(accv, out2.at[wid])
        accv[...] = zero16

        plsc.subcore_barrier()

        def cchunk(k, _):
            r0 = (wid * nchC_w + k) * CHM_ROWS
            pltpu.sync_copy(b_h.at[pl.ds(r0 * ROW, CHM)], bbuf0)
            mor = scan_chunk(bbuf0, CHM_ROWS)

            @pl.when(jnp.any(mor))
            def _():
                def rb(r, _):
                    off = r * ROW

                    def sc8(q, m):
                        bv = bbuf0[pl.ds(off + q * 16, 16)]
                        return jnp.logical_or(m, bv == v_vec)

                    mr = lax.fori_loop(0, ROW // 16, sc8, false16)

                    @pl.when(jnp.any(mr))
                    def _():
                        pltpu.sync_copy(a2_h.at[r0 + r], srcrow.at[0])
                        pltpu.sync_copy(em2_sh.at[srcrow.at[0]], crow.at[0])
                        accv[...] = accv[...] + masked_rowsum(bbuf0, off, crow)

                    return 0

                lax.fori_loop(0, CHM_ROWS, rb, 0)

            return 0

        lax.fori_loop(0, nchC_w, cchunk, 0)
        pltpu.sync_copy(accv, outs.at[wid])

    sds = jax.ShapeDtypeStruct
    return pl.kernel(
        body,
        out_type=(sds((NW, 16), f32), sds((NW, 16), f32), sds((NW, 16), f32)),
        mesh=mesh,
        compiler_params=pltpu.CompilerParams(needs_layout_passes=False),
        scratch_types=[
            pltpu.VMEM((CHA,), i32),
            pltpu.VMEM((CHA,), i32),
            pltpu.VMEM((1, ROW), i32),
            pltpu.VMEM((1, ROW), f32),
            pltpu.VMEM((CHM_ROWS, ROW), i32),
            pltpu.VMEM((CHM,), i32),
            pltpu.VMEM((CHM,), f32),
            pltpu.VMEM((CHM,), f32),
            pltpu.VMEM((CHM_ROWS, ROW), i32),
            pltpu.VMEM((CHM,), i32),
            pltpu.VMEM((CHM,), f32),
            pltpu.VMEM((CHM,), f32),
            pltpu.VMEM((ZC,), f32),
            pltpu.VMEM((16,), i32),
            pltpu.VMEM((16,), f32),
            pltpu.SemaphoreType.DMA,
            pltpu.SemaphoreType.DMA,
            pltpu.SemaphoreType.DMA,
            pltpu.SemaphoreType.DMA,
            pltpu.SemaphoreType.DMA,
            pltpu.VMEM_SHARED((Npad,), f32),
        ],
    )


def _tc_body(p1, p2, ps, wv, w2, w3, e1r, out):
    c1v = jnp.sum(p1[...])
    c2v = jnp.sum(p2[...])
    s = jnp.sum(ps[...])
    row = s * wv[...]
    h = jnp.maximum(jnp.dot(row, w2[...], preferred_element_type=f32), 0.0)
    v3 = jnp.dot(h, w3[...], preferred_element_type=f32)
    out[...] = v3 + e1r[...] + jnp.maximum(c1v, c2v)


def kernel(x, c, notj, edge_index, e1, edge_index_P, index_m, PT,
           W_v, W_v2, W_v3, v, m):
    N = c.shape[0]
    E = edge_index.shape[1]
    EP = edge_index_P.shape[1]
    EM = index_m.shape[0]

    mesh = plsc.VectorSubcoreMesh(core_axis_name="c", subcore_axis_name="s")
    NC, NS = mesh.num_cores, mesh.num_subcores
    NW = NC * NS

    def pad_pair(src, dst, tot):
        totp = -(-tot // CHA) * CHA
        if totp != tot:
            padn = totp - tot
            src = jnp.concatenate([src, jnp.zeros((padn,), i32)])
            dst = jnp.concatenate([dst, jnp.full((padn,), -1, i32)])
        return src, dst, totp

    srcE, dstE, Ep = pad_pair(edge_index[0], edge_index[1], E)
    srcP, dstP, EPp = pad_pair(edge_index_P[0], edge_index_P[1], EP)

    per_m = CHM * NW
    EMp = -(-EM // per_m) * per_m
    MR = EMp // ROW
    padn = EMp - EM
    a = index_m[:, 0]
    b = index_m[:, 1]
    pt = PT[:, 0]
    if padn:
        a = jnp.concatenate([a, jnp.zeros((padn,), i32)])
        b = jnp.concatenate([b, jnp.full((padn,), -1, i32)])
        pt = jnp.concatenate([pt, jnp.zeros((padn,), f32)])
    a2 = a.reshape(MR, ROW)

    v16 = jnp.full((16,), v, i32)
    cflat = c[:, 0]

    p1, p2, ps = _sc_call(N, Ep, EPp, MR, NC, NS, mesh)(
        cflat, dstE, srcE, dstP, srcP, a2, b, pt, v16)

    q = pl.pallas_call(
        _tc_body,
        out_shape=jax.ShapeDtypeStruct((1, 1), f32),
    )(p1, p2, ps, W_v, W_v2, W_v3, e1.reshape(1, 1))
    return q.reshape(1)


# speedup vs baseline: 588.5812x; 1.8091x over previous
import jax
import jax.numpy as jnp
from jax import lax
from jax.experimental import pallas as pl
from jax.experimental.pallas import tpu as pltpu
from jax.experimental.pallas import tpu_sc as plsc

f32 = jnp.float32
i32 = jnp.int32

ROW = 128
CHA_ROWS = 100
CHA = CHA_ROWS * ROW
CHM_ROWS = 32
CHM = CHM_ROWS * ROW
ZC = 1600


def _sc_call(N, Ep, EPp, MR, NC, NS, mesh):
    NW = NC * NS
    nchE = Ep // CHA
    nchP = EPp // CHA
    pairsE = (-(-nchE // NW) + 1) // 2
    pairsP = (-(-nchP // NW) + 1) // 2
    nchB_t = MR // CHM_ROWS // NS
    nchC_w = MR // CHM_ROWS // NW
    stripe0 = -(-N // NS)
    stripe = -(-stripe0 // ZC) * ZC
    Npad = NS * stripe
    nz = stripe // ZC

    def body(c_hbm, eiE_h, eiP_h, a2_h, b_h, pt_h, v_h,
             out1, out2, outs,
             dbuf0, dbuf1, srcrow, crow,
             abuf0, bbuf0, ptbuf0, msgbuf0,
             abuf1, bbuf1, ptbuf1, msgbuf1,
             zf, vbuf, accv,
             semA, semB, semS, semM0, semM1, em2_sh):
        cid = lax.axis_index("c")
        sid = lax.axis_index("s")
        wid = sid * NC + cid

        pltpu.sync_copy(v_h, vbuf)
        v_vec = vbuf[...]
        zero16 = jnp.zeros((16,), f32)
        false16 = jnp.zeros((16,), jnp.bool_)

        def zb(i, _):
            zf[pl.ds(i * 16, 16)] = zero16
            return 0

        lax.fori_loop(0, ZC // 16, zb, 0)
        for i in range(nz):
            pltpu.sync_copy(zf, em2_sh.at[pl.ds(sid * stripe + i * ZC, ZC)])
        plsc.subcore_barrier()

        def scan_chunk(buf, nrows):
            def it(r, mor):
                off = r * ROW
                for q in range(ROW // 16):
                    dv = buf[pl.ds(off + q * 16, 16)]
                    mor = jnp.logical_or(mor, dv == v_vec)
                return mor

            return lax.fori_loop(0, nrows, it, false16)

        def masked_rowsum(buf, off, valrow):
            def acc8(q, s):
                dv = buf[pl.ds(off + q * 16, 16)]
                cv = valrow[0, pl.ds(q * 16, 16)]
                return s + jnp.where(dv == v_vec, cv, 0.0)

            return lax.fori_loop(0, ROW // 16, acc8, zero16)

        bsets = ((abuf0, bbuf0, ptbuf0, msgbuf0, semM0),
                 (abuf1, bbuf1, ptbuf1, msgbuf1, semM1))

        def b_copies(k, st):
            ab, bb, ptb, _, sm = st
            r0 = (sid * nchB_t + k) * CHM_ROWS
            return ((a2_h.at[pl.ds(r0, CHM_ROWS)], ab, sm),
                    (b_h.at[pl.ds(r0 * ROW, CHM)], bb, sm),
                    (pt_h.at[pl.ds(r0 * ROW, CHM)], ptb, sm))

        def drain_scatters(st):
            ab, _, ptb, _, _ = st
            for r in range(CHM_ROWS):
                pltpu.make_async_copy(ptb.at[pl.ds(r * ROW, ROW)],
                                      em2_sh.at[ab.at[r]], semS).wait()

        for c3 in b_copies(0, bsets[0]):
            pltpu.async_copy(*c3)

        def bpair(tp, _):
            for par in range(2):
                cur = bsets[par]
                oth = bsets[1 - par]
                k = 2 * tp + par
                for c3 in b_copies(k, cur):
                    pltpu.make_async_copy(*c3).wait()

                @pl.when(k >= 1)
                def _(oth=oth):
                    drain_scatters(oth)

                @pl.when(k + 1 < nchB_t)
                def _(k=k, oth=oth):
                    for c3 in b_copies(k + 1, oth):
                        pltpu.async_copy(*c3)

                ab, bb, ptb, mb, _ = cur
                anym = jnp.any(scan_chunk(bb, CHM_ROWS))

                @pl.when(anym)
                def _(ab=ab, bb=bb, ptb=ptb, mb=mb):
                    def mit(i, _):
                        bv = bb[pl.ds(i * 16, 16)]
                        pv = ptb[pl.ds(i * 16, 16)]
                        mb[pl.ds(i * 16, 16)] = jnp.where(bv == v_vec, 0.0, pv)
                        return 0

                    lax.fori_loop(0, CHM // 16, mit, 0)
                    for r in range(CHM_ROWS):
                        pltpu.async_copy(mb.at[pl.ds(r * ROW, ROW)],
                                         em2_sh.at[ab.at[r]], semS, add=True)

                @pl.when(jnp.logical_not(anym))
                def _(ab=ab, ptb=ptb):
                    for r in range(CHM_ROWS):
                        pltpu.async_copy(ptb.at[pl.ds(r * ROW, ROW)],
                                         em2_sh.at[ab.at[r]], semS, add=True)

            return 0

        with jax.named_scope("phaseB"):
            lax.fori_loop(0, nchB_t // 2, bpair, 0)
            drain_scatters(bsets[1])

        accv[...] = zero16

        def slow_rows(ei_h, buf, ch):
            def rb(r, _):
                off = r * ROW

                def sc8(q, m):
                    dv = buf[pl.ds(off + q * 16, 16)]
                    return jnp.logical_or(m, dv == v_vec)

                mor = lax.fori_loop(0, ROW // 16, sc8, false16)

                @pl.when(jnp.any(mor))
                def _():
                    g = ch * CHA + off
                    pltpu.sync_copy(ei_h.at[0, pl.ds(g, ROW)], srcrow.at[0])
                    pltpu.sync_copy(c_hbm.at[srcrow.at[0]], crow.at[0])
                    accv[...] = accv[...] + masked_rowsum(buf, off, crow)

                return 0

            lax.fori_loop(0, CHA_ROWS, rb, 0)

        def edge_phase(ei_h, nch, pairs):
            def sl(ch):
                return ei_h.at[1, pl.ds(ch * CHA, CHA)]

            pltpu.async_copy(sl(wid), dbuf0, semA)
            bufs = ((dbuf0, semA), (dbuf1, semB))

            def pair(tp, _):
                for par in range(2):
                    buf, sem = bufs[par]
                    obuf, osem = bufs[1 - par]
                    t = 2 * tp + par
                    ch = wid + NW * t

                    @pl.when(ch < nch)
                    def _(buf=buf, sem=sem, obuf=obuf, osem=osem, ch=ch):
                        pltpu.make_async_copy(sl(ch), buf, sem).wait()
                        chn = ch + NW

                        @pl.when(chn < nch)
                        def _():
                            pltpu.async_copy(sl(chn), obuf, osem)

                        mor = scan_chunk(buf, CHA_ROWS)

                        @pl.when(jnp.any(mor))
                        def _():
                            slow_rows(ei_h, buf, ch)

                return 0

            lax.fori_loop(0, pairs, pair, 0)

        with jax.named_scope("phaseA"):
            edge_phase(eiE_h, nchE, pairsE)
            pltpu.sync_copy(accv, out1.at[wid])
            accv[...] = zero16

        with jax.named_scope("phaseA2"):
            edge_phase(eiP_h, nchP, pairsP)
            pltpu.sync_copy(accv, out2.at[wid])
            accv[...] = zero16

        plsc.subcore_barrier()

        def cchunk(k, _):
            r0 = (wid * nchC_w + k) * CHM_ROWS
            pltpu.sync_copy(b_h.at[pl.ds(r0 * ROW, CHM)], bbuf0)
            mor = scan_chunk(bbuf0, CHM_ROWS)

            @pl.when(jnp.any(mor))
            def _():
                def rb(r, _):
                    off = r * ROW

                    def sc8(q, m):
                        bv = bbuf0[pl.ds(off + q * 16, 16)]
                        return jnp.logical_or(m, bv == v_vec)

                    mr = lax.fori_loop(0, ROW // 16, sc8, false16)

                    @pl.when(jnp.any(mr))
                    def _():
                        pltpu.sync_copy(a2_h.at[r0 + r], srcrow.at[0])
                        pltpu.sync_copy(em2_sh.at[srcrow.at[0]], crow.at[0])
                        accv[...] = accv[...] + masked_rowsum(bbuf0, off, crow)

                    return 0

                lax.fori_loop(0, CHM_ROWS, rb, 0)

            return 0

        with jax.named_scope("phaseC"):
            lax.fori_loop(0, nchC_w, cchunk, 0)
            pltpu.sync_copy(accv, outs.at[wid])

    sds = jax.ShapeDtypeStruct
    return pl.kernel(
        body,
        out_type=(sds((NW, 16), f32), sds((NW, 16), f32), sds((NW, 16), f32)),
        mesh=mesh,
        compiler_params=pltpu.CompilerParams(needs_layout_passes=False),
        scratch_types=[
            pltpu.VMEM((CHA,), i32),
            pltpu.VMEM((CHA,), i32),
            pltpu.VMEM((1, ROW), i32),
            pltpu.VMEM((1, ROW), f32),
            pltpu.VMEM((CHM_ROWS, ROW), i32),
            pltpu.VMEM((CHM,), i32),
            pltpu.VMEM((CHM,), f32),
            pltpu.VMEM((CHM,), f32),
            pltpu.VMEM((CHM_ROWS, ROW), i32),
            pltpu.VMEM((CHM,), i32),
            pltpu.VMEM((CHM,), f32),
            pltpu.VMEM((CHM,), f32),
            pltpu.VMEM((ZC,), f32),
            pltpu.VMEM((16,), i32),
            pltpu.VMEM((16,), f32),
            pltpu.SemaphoreType.DMA,
            pltpu.SemaphoreType.DMA,
            pltpu.SemaphoreType.DMA,
            pltpu.SemaphoreType.DMA,
            pltpu.SemaphoreType.DMA,
            pltpu.VMEM_SHARED((Npad,), f32),
        ],
    )


def _tc_body(p1, p2, ps, wv, w2, w3, e1r, out):
    c1v = jnp.sum(p1[...])
    c2v = jnp.sum(p2[...])
    s = jnp.sum(ps[...])
    row = s * wv[...]
    h = jnp.maximum(jnp.dot(row, w2[...], preferred_element_type=f32), 0.0)
    v3 = jnp.dot(h, w3[...], preferred_element_type=f32)
    out[...] = v3 + e1r[...] + jnp.maximum(c1v, c2v)


def kernel(x, c, notj, edge_index, e1, edge_index_P, index_m, PT,
           W_v, W_v2, W_v3, v, m):
    N = c.shape[0]
    E = edge_index.shape[1]
    EP = edge_index_P.shape[1]
    EM = index_m.shape[0]

    mesh = plsc.VectorSubcoreMesh(core_axis_name="c", subcore_axis_name="s")
    NC, NS = mesh.num_cores, mesh.num_subcores
    NW = NC * NS

    def pad_ei(ei, tot):
        totp = -(-tot // CHA) * CHA
        if totp != tot:
            padn = totp - tot
            pad = jnp.stack([jnp.zeros((padn,), i32),
                             jnp.full((padn,), -1, i32)])
            ei = jnp.concatenate([ei, pad], axis=1)
        return ei, totp

    eiE, Ep = pad_ei(edge_index, E)
    eiP, EPp = pad_ei(edge_index_P, EP)

    per_m = CHM * NW
    EMp = -(-EM // per_m) * per_m
    MR = EMp // ROW
    padn = EMp - EM
    a = index_m[:, 0]
    b = index_m[:, 1]
    pt = PT[:, 0]
    if padn:
        a = jnp.concatenate([a, jnp.zeros((padn,), i32)])
        b = jnp.concatenate([b, jnp.full((padn,), -1, i32)])
        pt = jnp.concatenate([pt, jnp.zeros((padn,), f32)])
    a2 = a.reshape(MR, ROW)

    v16 = jnp.full((16,), v, i32)
    cflat = c[:, 0]

    p1, p2, ps = _sc_call(N, Ep, EPp, MR, NC, NS, mesh)(
        cflat, eiE, eiP, a2, b, pt, v16)

    q = pl.pallas_call(
        _tc_body,
        out_shape=jax.ShapeDtypeStruct((1, 1), f32),
    )(p1, p2, ps, W_v, W_v2, W_v3, e1.reshape(1, 1))
    return q.reshape(1)
